# baseline (speedup 1.0000x reference)
import functools

import jax
import jax.numpy as jnp
import numpy as np
from jax.experimental import pallas as pl
from jax.experimental.pallas import tpu as pltpu
from jax.experimental.pallas import tpu_sc as plsc

_NPOINT1, _RADIUS1, _NSAMPLE1 = 1024, 0.2, 32
_NPOINT2, _RADIUS2, _NSAMPLE2 = 256, 0.4, 32
_ALPHA = 0.2
_BN_S = 1.0 / np.sqrt(1.0 + 1e-5)
_BIG = 1e10


def _feat_body(nw, combine, x_ref, *refs):
    w_refs = refs[:nw]
    b_refs = refs[nw:2 * nw]
    a_ref = refs[2 * nw]
    x = x_ref[0]
    h = x
    for i in range(nw):
        h = jax.nn.relu((jnp.dot(h, w_refs[i][...]) + b_refs[i][...]) * _BN_S)
    feat = h
    xyz = x[:, :3]
    ga = jnp.dot(xyz, a_ref[:3, :]) + jnp.dot(feat, a_ref[3:, :])
    if combine:
        v_ref = refs[2 * nw + 1]
        f = feat.shape[-1]
        v_ref[0, :, 0:f] = feat
        v_ref[0, :, f:2 * f] = ga
    else:
        refs[2 * nw + 1][0] = feat
        refs[2 * nw + 2][0] = ga


def _feat_table(x, Ws, bs, a, combine):
    B, N, Cin = x.shape
    nw = len(Ws)
    f = Ws[-1].shape[1]
    full = lambda s: pl.BlockSpec(s, lambda b: (0,) * len(s))
    in_specs = [pl.BlockSpec((1, N, Cin), lambda b: (b, 0, 0))]
    in_specs += [full(W.shape) for W in Ws]
    in_specs += [full((1, bb.shape[0])) for bb in bs]
    in_specs += [full(a.shape)]
    if combine:
        out_specs = pl.BlockSpec((1, N, 2 * f), lambda b: (b, 0, 0))
        out_shape = jax.ShapeDtypeStruct((B, N, 2 * f), x.dtype)
    else:
        out_specs = [pl.BlockSpec((1, N, f), lambda b: (b, 0, 0))] * 2
        out_shape = [jax.ShapeDtypeStruct((B, N, f), x.dtype)] * 2
    out = pl.pallas_call(
        functools.partial(_feat_body, nw, combine),
        grid=(B,),
        in_specs=in_specs,
        out_specs=out_specs,
        out_shape=out_shape,
    )(x, *Ws, *[bb.reshape(1, -1) for bb in bs], a)
    return out


def _fps_body(npoint, n, xt_ref, idx_ref, cxyz_ref):
    B = xt_ref.shape[0]
    tboff = jax.lax.broadcasted_iota(jnp.int32, (1, B), 1) * n
    liota = jax.lax.broadcasted_iota(jnp.int32, (B, n), 1)

    def body(i, carry):
        distance, far = carry
        idx_ref[pl.ds(i, 1), :] = jnp.transpose(far, (1, 0)) + tboff
        ohf = jnp.where(liota == far, 1.0, 0.0)
        dist = jnp.zeros_like(distance)
        ccs = []
        for c in range(3):
            xc = xt_ref[:, c, :]
            cc = jnp.sum(xc * ohf, axis=1, keepdims=True)
            ccs.append(cc)
            t = xc - cc
            dist = dist + t * t
        cxyz_ref[pl.ds(i, 1), :, :] = jnp.concatenate(ccs, axis=1)[None]
        distance = jnp.minimum(dist, distance)
        far = jnp.argmax(distance, axis=1).astype(jnp.int32)[:, None]
        return distance, far

    distance0 = jnp.full((B, n), _BIG, jnp.float32)
    far0 = jnp.zeros((B, 1), jnp.int32)
    jax.lax.fori_loop(0, npoint, body, (distance0, far0))


def _fps(xyzT, npoint):
    B, _, N = xyzT.shape
    idxT, cxyzP = pl.pallas_call(
        functools.partial(_fps_body, npoint, N),
        in_specs=[pl.BlockSpec((B, 3, N), lambda: (0, 0, 0))],
        out_specs=[pl.BlockSpec((npoint, B), lambda: (0, 0)),
                   pl.BlockSpec((npoint, B, 3), lambda: (0, 0, 0))],
        out_shape=[jax.ShapeDtypeStruct((npoint, B), jnp.int32),
                   jax.ShapeDtypeStruct((npoint, B, 3), jnp.float32)],
    )(xyzT)
    return jnp.transpose(idxT, (1, 0)), jnp.transpose(cxyzP, (1, 0, 2))


def _ballq_body(r2, nsample, n, cxyz_ref, xt_ref, gi_ref):
    b = pl.program_id(0)
    cb = cxyz_ref.shape[1]
    c = cxyz_ref[0]
    xt = xt_ref[0]
    d = -2.0 * jnp.dot(c, xt)
    d = d + jnp.sum(c * c, axis=-1)[:, None]
    xn = xt[0] * xt[0] + xt[1] * xt[1] + xt[2] * xt[2]
    d = d + xn[None, :]
    dm0 = jnp.where(d > r2, _BIG, d)
    cnt = jnp.sum(jnp.where(d > r2, 0, 1), axis=1)
    liota = jax.lax.broadcasted_iota(jnp.int32, (cb, n), 1)
    siota = jax.lax.broadcasted_iota(jnp.int32, (cb, nsample), 1)
    first = jnp.argmin(dm0, axis=1).astype(jnp.int32)
    out0 = jnp.broadcast_to(first[:, None], (cb, nsample))

    def body(k, carry):
        dm, out, prev = carry
        dm = jnp.where(liota == prev[:, None], _BIG, dm)
        idx = jnp.argmin(dm, axis=1).astype(jnp.int32)
        sel = jnp.where(k < cnt, idx, first)
        out = jnp.where(siota == k, sel[:, None], out)
        return dm, out, idx

    _, out, _ = jax.lax.fori_loop(1, nsample, body, (dm0, out0, first))
    gi_ref[0] = out + b * n


def _ballq(cxyz, xyzT, radius, nsample, cb):
    B, npnt, _ = cxyz.shape
    N = xyzT.shape[2]
    gi = pl.pallas_call(
        functools.partial(_ballq_body, radius ** 2, nsample, N),
        grid=(B, npnt // cb),
        in_specs=[pl.BlockSpec((1, cb, 3), lambda b, j: (b, j, 0)),
                  pl.BlockSpec((1, 3, N), lambda b, j: (b, 0, 0))],
        out_specs=pl.BlockSpec((1, cb, nsample), lambda b, j: (b, j, 0)),
        out_shape=jax.ShapeDtypeStruct((B, npnt, nsample), jnp.int32),
    )(cxyz, xyzT)
    return gi


def _sc_gather_impl(table, idx, window):
    K = idx.shape[0]
    Wd = table.shape[1]
    mesh = plsc.VectorSubcoreMesh(core_axis_name="core",
                                  subcore_axis_name="subcore")

    @pl.kernel(out_type=jax.ShapeDtypeStruct((K, Wd), table.dtype), mesh=mesh)
    def kern(x_hbm, i_hbm, o_hbm):
        def body(i_vmem, o_vmem):
            pltpu.sync_copy(x_hbm.at[i_vmem.at[0]], o_vmem)

        pltpu.emit_pipeline(
            body,
            grid=(K // window,),
            in_specs=[pl.BlockSpec((1, window), index_map=lambda i: (0, i))],
            out_specs=[pl.BlockSpec((window, Wd), index_map=lambda i: (i, 0))],
            core_axis_name=("core", "subcore"),
            dimension_semantics=(pltpu.PARALLEL,),
        )(i_hbm, o_hbm)

    return kern(table, idx.reshape(1, K))


_gather = _sc_gather_impl


def _gather_rows(table, idx):
    k = idx.shape[0]
    pad = (-k) % 4096
    if pad:
        idx = jnp.concatenate([idx, jnp.zeros((pad,), jnp.int32)])
    out = _gather(table, idx, 128)
    return out[:k] if pad else out


def _att_math(gfeat, ga, ca, cxyz, out_ref):
    l = ga - ca[:, None, :]
    l = jnp.where(l >= 0, l, _ALPHA * l)
    m = jnp.max(l, axis=1, keepdims=True)
    e = jnp.exp(l - m)
    s = jnp.sum(e, axis=1, keepdims=True)
    att = e / s
    feats = jnp.sum(att * gfeat, axis=1)
    out_ref[0, :, 0:3] = cxyz
    out_ref[0, :, 3:] = feats


def _att_body(f, gfeat_ref, ga_ref, ca_ref, cxyz_ref, out_ref):
    _att_math(gfeat_ref[0], ga_ref[0], ca_ref[0], cxyz_ref[0], out_ref)


def _att_comb_body(f, grp_ref, cent_ref, cxyz_ref, out_ref):
    g = grp_ref[0]
    cent = cent_ref[0]
    _att_math(g[:, :, 0:f], g[:, :, f:2 * f], cent[:, f:2 * f],
              cxyz_ref[0], out_ref)


def _att(gfeat, ga, ca, cxyz, f, cb):
    B, npnt, S, _ = gfeat.shape
    out = pl.pallas_call(
        functools.partial(_att_body, f),
        grid=(B, npnt // cb),
        in_specs=[pl.BlockSpec((1, cb, S, f), lambda b, j: (b, j, 0, 0)),
                  pl.BlockSpec((1, cb, S, f), lambda b, j: (b, j, 0, 0)),
                  pl.BlockSpec((1, cb, f), lambda b, j: (b, j, 0)),
                  pl.BlockSpec((1, cb, 3), lambda b, j: (b, j, 0))],
        out_specs=pl.BlockSpec((1, cb, 3 + f), lambda b, j: (b, j, 0)),
        out_shape=jax.ShapeDtypeStruct((B, npnt, 3 + f), jnp.float32),
    )(gfeat, ga, ca, cxyz)
    return out


def _att_comb(grp, cent, cxyz, f, cb):
    B, npnt, S, _ = grp.shape
    out = pl.pallas_call(
        functools.partial(_att_comb_body, f),
        grid=(B, npnt // cb),
        in_specs=[pl.BlockSpec((1, cb, S, 2 * f), lambda b, j: (b, j, 0, 0)),
                  pl.BlockSpec((1, cb, 2 * f), lambda b, j: (b, j, 0)),
                  pl.BlockSpec((1, cb, 3), lambda b, j: (b, j, 0))],
        out_specs=pl.BlockSpec((1, cb, 3 + f), lambda b, j: (b, j, 0)),
        out_shape=jax.ShapeDtypeStruct((B, npnt, 3 + f), jnp.float32),
    )(grp, cent, cxyz)
    return out


def _knn3_mean(qxyz, sxyzT, sfeat):
    nq = qxyz.shape[0]
    ns = sxyzT.shape[1]
    d = -2.0 * jnp.dot(qxyz, sxyzT)
    d = d + jnp.sum(qxyz * qxyz, axis=-1)[:, None]
    sn = sxyzT[0] * sxyzT[0] + sxyzT[1] * sxyzT[1] + sxyzT[2] * sxyzT[2]
    d = d + sn[None, :]
    liota = jax.lax.broadcasted_iota(jnp.int32, (nq, ns), 1)
    wm = jnp.zeros((nq, ns), jnp.float32)
    for _ in range(3):
        idx = jnp.argmin(d, axis=1).astype(jnp.int32)[:, None]
        wm = jnp.where(liota == idx, 1.0, wm)
        d = jnp.where(liota == idx, _BIG, d)
    return jnp.dot(wm, sfeat) / 3.0


def _fp_body(nw, fq, q_ref, st_ref, sf_ref, *refs):
    w_refs = refs[:nw]
    b_refs = refs[nw:2 * nw]
    out_ref = refs[2 * nw]
    q = q_ref[0]
    qxyz = q[:, :3]
    qfeat = q[:, 3:3 + fq]
    interp = _knn3_mean(qxyz, st_ref[0], sf_ref[0])
    w0 = w_refs[0][...]
    h = jax.nn.relu((jnp.dot(qfeat, w0[:fq, :]) + jnp.dot(interp, w0[fq:, :])
                     + b_refs[0][...]) * _BN_S)
    for i in range(1, nw):
        h = jax.nn.relu((jnp.dot(h, w_refs[i][...]) + b_refs[i][...]) * _BN_S)
    out_ref[0, :, 0:3] = qxyz
    out_ref[0, :, 3:] = h


def _fp(q, sxyzT, sfeat, Ws, bs, fq, rb):
    B, Nq, Cq = q.shape
    Ns = sfeat.shape[1]
    Fs = sfeat.shape[2]
    nw = len(Ws)
    fout = Ws[-1].shape[1]
    full = lambda s: pl.BlockSpec(s, lambda b, j: (0,) * len(s))
    in_specs = [pl.BlockSpec((1, rb, Cq), lambda b, j: (b, j, 0)),
                pl.BlockSpec((1, 3, Ns), lambda b, j: (b, 0, 0)),
                pl.BlockSpec((1, Ns, Fs), lambda b, j: (b, 0, 0))]
    in_specs += [full(W.shape) for W in Ws]
    in_specs += [full((1, bb.shape[0])) for bb in bs]
    out = pl.pallas_call(
        functools.partial(_fp_body, nw, fq),
        grid=(B, Nq // rb),
        in_specs=in_specs,
        out_specs=pl.BlockSpec((1, rb, 3 + fout), lambda b, j: (b, j, 0)),
        out_shape=jax.ShapeDtypeStruct((B, Nq, 3 + fout), jnp.float32),
    )(q, sxyzT, sfeat, *Ws, *[bb.reshape(1, -1) for bb in bs])
    return out


def _fp_head_body(nw, fq, q_ref, st_ref, sf_ref, *refs):
    w_refs = refs[:nw]
    b_refs = refs[nw:2 * nw]
    hw_ref = refs[2 * nw]
    hb_ref = refs[2 * nw + 1]
    out_ref = refs[2 * nw + 2]
    q = q_ref[0]
    qxyz = q[:, :3]
    qfeat = q[:, 3:3 + fq]
    interp = _knn3_mean(qxyz, st_ref[0], sf_ref[0])
    w0 = w_refs[0][...]
    h = jax.nn.relu((jnp.dot(qfeat, w0[:fq, :]) + jnp.dot(interp, w0[fq:, :])
                     + b_refs[0][...]) * _BN_S)
    for i in range(1, nw):
        h = jax.nn.relu((jnp.dot(h, w_refs[i][...]) + b_refs[i][...]) * _BN_S)
    logits = jnp.dot(h, hw_ref[...]) + hb_ref[...]
    m = jnp.max(logits, axis=1, keepdims=True)
    sh = logits - m
    out_ref[0] = sh - jnp.log(jnp.sum(jnp.exp(sh), axis=1, keepdims=True))


def _fp_head(q, sxyzT, sfeat, Ws, bs, headW, headb, fq, rb):
    B, Nq, Cq = q.shape
    Ns = sfeat.shape[1]
    Fs = sfeat.shape[2]
    nw = len(Ws)
    ncls = headW.shape[1]
    full = lambda s: pl.BlockSpec(s, lambda b, j: (0,) * len(s))
    in_specs = [pl.BlockSpec((1, rb, Cq), lambda b, j: (b, j, 0)),
                pl.BlockSpec((1, 3, Ns), lambda b, j: (b, 0, 0)),
                pl.BlockSpec((1, Ns, Fs), lambda b, j: (b, 0, 0))]
    in_specs += [full(W.shape) for W in Ws]
    in_specs += [full((1, bb.shape[0])) for bb in bs]
    in_specs += [full(headW.shape), full((1, ncls))]
    out = pl.pallas_call(
        functools.partial(_fp_head_body, nw, fq),
        grid=(B, Nq // rb),
        in_specs=in_specs,
        out_specs=pl.BlockSpec((1, rb, ncls), lambda b, j: (b, j, 0)),
        out_shape=jax.ShapeDtypeStruct((B, Nq, ncls), jnp.float32),
    )(q, sxyzT, sfeat, *Ws, *[bb.reshape(1, -1) for bb in bs], headW,
      headb.reshape(1, -1))
    return out


def _gac_layer(x, Ws, bs, a, npoint, radius, nsample, cb_ball, cb_att,
               xyzT=None):
    B, N, _ = x.shape
    f = Ws[-1].shape[1]
    combine = 2 * f <= 256
    tabs = _feat_table(x, Ws, bs, a, combine)
    if xyzT is None:
        xyzT = jnp.transpose(x[..., :3], (0, 2, 1))
    ci, cxyz = _fps(xyzT, npoint)
    gi = _ballq(cxyz, xyzT, radius, nsample, cb_ball)
    gif = gi.reshape(-1)
    cif = ci.reshape(-1)
    if combine:
        tabf = tabs.reshape(B * N, 2 * f)
        grp = _gather_rows(tabf, gif).reshape(B, npoint, nsample, 2 * f)
        cent = _gather_rows(tabf, cif).reshape(B, npoint, 2 * f)
        return _att_comb(grp, cent, cxyz, f, cb_att), cxyz
    feat, ga = tabs
    featf = feat.reshape(B * N, f)
    gaf = ga.reshape(B * N, f)
    gfeat = _gather_rows(featf, gif).reshape(B, npoint, nsample, f)
    gga = _gather_rows(gaf, gif).reshape(B, npoint, nsample, f)
    ca = _gather_rows(gaf, cif).reshape(B, npoint, f)
    return _att(gfeat, gga, ca, cxyz, f, cb_att), cxyz


def kernel(points, params):
    B = points.shape[0]
    l1, cxyz1 = _gac_layer(points, params['l1_W'], params['l1_b'],
                           params['a1'], _NPOINT1, _RADIUS1, _NSAMPLE1,
                           cb_ball=8, cb_att=128)
    l2, _ = _gac_layer(l1, params['l2_W'], params['l2_b'], params['a2'],
                       _NPOINT2, _RADIUS2, _NSAMPLE2, cb_ball=8, cb_att=128,
                       xyzT=jnp.transpose(cxyz1, (0, 2, 1)))
    l2xyzT = jnp.transpose(l2[..., :3], (0, 2, 1))
    l1u = _fp(l1, l2xyzT, l2[..., 3:], params['fp2_W'], params['fp2_b'],
              fq=128, rb=1024)
    l1uxyzT = jnp.transpose(l1u[..., :3], (0, 2, 1))
    logits = _fp_head(points, l1uxyzT, l1u[..., 3:], params['fp1_W'],
                      params['fp1_b'], params['head_W'], params['head_b'],
                      fq=6, rb=1024)
    return logits


# speedup vs baseline: 3.1064x; 3.1064x over previous
import functools

import jax
import jax.numpy as jnp
import numpy as np
from jax.experimental import pallas as pl
from jax.experimental.pallas import tpu as pltpu
from jax.experimental.pallas import tpu_sc as plsc

_NPOINT1, _RADIUS1, _NSAMPLE1 = 1024, 0.2, 32
_NPOINT2, _RADIUS2, _NSAMPLE2 = 256, 0.4, 32
_ALPHA = 0.2
_BN_S = 1.0 / np.sqrt(1.0 + 1e-5)
_BIG = 1e10


def _feat_body(nw, combine, x_ref, *refs):
    w_refs = refs[:nw]
    b_refs = refs[nw:2 * nw]
    a_ref = refs[2 * nw]
    x = x_ref[0]
    h = x
    for i in range(nw):
        h = jax.nn.relu((jnp.dot(h, w_refs[i][...]) + b_refs[i][...]) * _BN_S)
    feat = h
    xyz = x[:, :3]
    ga = jnp.dot(xyz, a_ref[:3, :]) + jnp.dot(feat, a_ref[3:, :])
    if combine:
        v_ref = refs[2 * nw + 1]
        f = feat.shape[-1]
        v_ref[0, :, 0:f] = feat
        v_ref[0, :, f:2 * f] = ga
    else:
        refs[2 * nw + 1][0] = feat
        refs[2 * nw + 2][0] = ga


def _feat_table(x, Ws, bs, a, combine):
    B, N, Cin = x.shape
    nw = len(Ws)
    f = Ws[-1].shape[1]
    full = lambda s: pl.BlockSpec(s, lambda b: (0,) * len(s))
    in_specs = [pl.BlockSpec((1, N, Cin), lambda b: (b, 0, 0))]
    in_specs += [full(W.shape) for W in Ws]
    in_specs += [full((1, bb.shape[0])) for bb in bs]
    in_specs += [full(a.shape)]
    if combine:
        out_specs = pl.BlockSpec((1, N, 2 * f), lambda b: (b, 0, 0))
        out_shape = jax.ShapeDtypeStruct((B, N, 2 * f), x.dtype)
    else:
        out_specs = [pl.BlockSpec((1, N, f), lambda b: (b, 0, 0))] * 2
        out_shape = [jax.ShapeDtypeStruct((B, N, f), x.dtype)] * 2
    out = pl.pallas_call(
        functools.partial(_feat_body, nw, combine),
        grid=(B,),
        in_specs=in_specs,
        out_specs=out_specs,
        out_shape=out_shape,
    )(x, *Ws, *[bb.reshape(1, -1) for bb in bs], a)
    return out


def _fps_body(npoint, n, xt_ref, idx_ref, cxyz_ref):
    B = xt_ref.shape[0]
    tboff = jax.lax.broadcasted_iota(jnp.int32, (1, B), 1) * n
    liota = jax.lax.broadcasted_iota(jnp.int32, (B, n), 1)

    def body(i, carry):
        distance, far = carry
        idx_ref[pl.ds(i, 1), :] = jnp.transpose(far, (1, 0)) + tboff
        ohf = jnp.where(liota == far, 1.0, 0.0)
        dist = jnp.zeros_like(distance)
        ccs = []
        for c in range(3):
            xc = xt_ref[:, c, :]
            cc = jnp.sum(xc * ohf, axis=1, keepdims=True)
            ccs.append(cc)
            t = xc - cc
            dist = dist + t * t
        cxyz_ref[pl.ds(i, 1), :, :] = jnp.concatenate(ccs, axis=1)[None]
        distance = jnp.minimum(dist, distance)
        far = jnp.argmax(distance, axis=1).astype(jnp.int32)[:, None]
        return distance, far

    distance0 = jnp.full((B, n), _BIG, jnp.float32)
    far0 = jnp.zeros((B, 1), jnp.int32)
    jax.lax.fori_loop(0, npoint, body, (distance0, far0))


def _fps(xyzT, npoint):
    B, _, N = xyzT.shape
    idxT, cxyzP = pl.pallas_call(
        functools.partial(_fps_body, npoint, N),
        in_specs=[pl.BlockSpec((B, 3, N), lambda: (0, 0, 0))],
        out_specs=[pl.BlockSpec((npoint, B), lambda: (0, 0)),
                   pl.BlockSpec((npoint, B, 3), lambda: (0, 0, 0))],
        out_shape=[jax.ShapeDtypeStruct((npoint, B), jnp.int32),
                   jax.ShapeDtypeStruct((npoint, B, 3), jnp.float32)],
    )(xyzT)
    return jnp.transpose(idxT, (1, 0)), jnp.transpose(cxyzP, (1, 0, 2))


def _ballq_body(r2, nsample, n, cxyz_ref, xt_ref, gi_ref, dm_ref):
    b = pl.program_id(0)
    cb = cxyz_ref.shape[1]
    c = cxyz_ref[0]
    xt = xt_ref[0]
    d = -2.0 * jnp.dot(c, xt)
    d = d + jnp.sum(c * c, axis=-1)[:, None]
    xn = xt[0] * xt[0] + xt[1] * xt[1] + xt[2] * xt[2]
    d = d + xn[None, :]
    dmv0 = jnp.where(d > r2, _BIG, d)
    dm_ref[...] = dmv0
    cnt = jnp.sum(jnp.where(d > r2, 0, 1), axis=1)
    liota = jax.lax.broadcasted_iota(jnp.int32, (cb, n), 1)
    siota = jax.lax.broadcasted_iota(jnp.int32, (cb, nsample), 1)
    first = jnp.argmin(dmv0, axis=1).astype(jnp.int32)
    gi_ref[0] = jnp.broadcast_to((first + b * n)[:, None], gi_ref.shape[1:])

    def body(k, prev):
        dmm = jnp.where(liota == prev[:, None], _BIG, dm_ref[...])
        dm_ref[...] = dmm
        idx = jnp.argmin(dmm, axis=1).astype(jnp.int32)
        sel = jnp.where(k < cnt, idx, first) + b * n
        gi_ref[0] = jnp.where(siota == k, sel[:, None], gi_ref[0])
        return idx

    jax.lax.fori_loop(1, nsample, body, first)


def _ballq(cxyz, xyzT, radius, nsample, cb):
    B, npnt, _ = cxyz.shape
    N = xyzT.shape[2]
    gi = pl.pallas_call(
        functools.partial(_ballq_body, radius ** 2, nsample, N),
        grid=(B, npnt // cb),
        in_specs=[pl.BlockSpec((1, cb, 3), lambda b, j: (b, j, 0)),
                  pl.BlockSpec((1, 3, N), lambda b, j: (b, 0, 0))],
        out_specs=pl.BlockSpec((1, cb, nsample), lambda b, j: (b, j, 0)),
        out_shape=jax.ShapeDtypeStruct((B, npnt, nsample), jnp.int32),
        scratch_shapes=[pltpu.VMEM((cb, N), jnp.float32)],
    )(cxyz, xyzT)
    return gi


def _sc_gather_impl(table, idx, window):
    K = idx.shape[0]
    Wd = table.shape[1]
    mesh = plsc.VectorSubcoreMesh(core_axis_name="core",
                                  subcore_axis_name="subcore")

    @pl.kernel(out_type=jax.ShapeDtypeStruct((K, Wd), table.dtype), mesh=mesh)
    def kern(x_hbm, i_hbm, o_hbm):
        def body(i_vmem, o_vmem):
            pltpu.sync_copy(x_hbm.at[i_vmem.at[0]], o_vmem)

        pltpu.emit_pipeline(
            body,
            grid=(K // window,),
            in_specs=[pl.BlockSpec((1, window), index_map=lambda i: (0, i))],
            out_specs=[pl.BlockSpec((window, Wd), index_map=lambda i: (i, 0))],
            core_axis_name=("core", "subcore"),
            dimension_semantics=(pltpu.PARALLEL,),
        )(i_hbm, o_hbm)

    return kern(table, idx.reshape(1, K))


_gather = _sc_gather_impl


def _gather_rows(table, idx):
    k = idx.shape[0]
    pad = (-k) % 4096
    if pad:
        idx = jnp.concatenate([idx, jnp.zeros((pad,), jnp.int32)])
    out = _gather(table, idx, 128)
    return out[:k] if pad else out


def _att_math(gfeat, ga, ca, cxyz, out_ref):
    l = ga - ca[:, None, :]
    l = jnp.where(l >= 0, l, _ALPHA * l)
    m = jnp.max(l, axis=1, keepdims=True)
    e = jnp.exp(l - m)
    s = jnp.sum(e, axis=1, keepdims=True)
    att = e / s
    feats = jnp.sum(att * gfeat, axis=1)
    out_ref[0, :, 0:3] = cxyz
    out_ref[0, :, 3:] = feats


def _att_body(f, gfeat_ref, ga_ref, ca_ref, cxyz_ref, out_ref):
    _att_math(gfeat_ref[0], ga_ref[0], ca_ref[0], cxyz_ref[0], out_ref)


def _att_comb_body(f, grp_ref, cent_ref, cxyz_ref, out_ref):
    g = grp_ref[0]
    cent = cent_ref[0]
    _att_math(g[:, :, 0:f], g[:, :, f:2 * f], cent[:, f:2 * f],
              cxyz_ref[0], out_ref)


def _att(gfeat, ga, ca, cxyz, f, cb):
    B, npnt, S, _ = gfeat.shape
    out = pl.pallas_call(
        functools.partial(_att_body, f),
        grid=(B, npnt // cb),
        in_specs=[pl.BlockSpec((1, cb, S, f), lambda b, j: (b, j, 0, 0)),
                  pl.BlockSpec((1, cb, S, f), lambda b, j: (b, j, 0, 0)),
                  pl.BlockSpec((1, cb, f), lambda b, j: (b, j, 0)),
                  pl.BlockSpec((1, cb, 3), lambda b, j: (b, j, 0))],
        out_specs=pl.BlockSpec((1, cb, 3 + f), lambda b, j: (b, j, 0)),
        out_shape=jax.ShapeDtypeStruct((B, npnt, 3 + f), jnp.float32),
    )(gfeat, ga, ca, cxyz)
    return out


def _att_comb(grp, cent, cxyz, f, cb):
    B, npnt, S, _ = grp.shape
    out = pl.pallas_call(
        functools.partial(_att_comb_body, f),
        grid=(B, npnt // cb),
        in_specs=[pl.BlockSpec((1, cb, S, 2 * f), lambda b, j: (b, j, 0, 0)),
                  pl.BlockSpec((1, cb, 2 * f), lambda b, j: (b, j, 0)),
                  pl.BlockSpec((1, cb, 3), lambda b, j: (b, j, 0))],
        out_specs=pl.BlockSpec((1, cb, 3 + f), lambda b, j: (b, j, 0)),
        out_shape=jax.ShapeDtypeStruct((B, npnt, 3 + f), jnp.float32),
    )(grp, cent, cxyz)
    return out


def _knn3_mean(qxyz, sxyzT, sfeat):
    nq = qxyz.shape[0]
    ns = sxyzT.shape[1]
    d = -2.0 * jnp.dot(qxyz, sxyzT)
    d = d + jnp.sum(qxyz * qxyz, axis=-1)[:, None]
    sn = sxyzT[0] * sxyzT[0] + sxyzT[1] * sxyzT[1] + sxyzT[2] * sxyzT[2]
    d = d + sn[None, :]
    liota = jax.lax.broadcasted_iota(jnp.int32, (nq, ns), 1)
    wm = jnp.zeros((nq, ns), jnp.float32)
    for _ in range(3):
        idx = jnp.argmin(d, axis=1).astype(jnp.int32)[:, None]
        wm = jnp.where(liota == idx, 1.0, wm)
        d = jnp.where(liota == idx, _BIG, d)
    return jnp.dot(wm, sfeat) / 3.0


def _fp_body(nw, fq, q_ref, st_ref, sf_ref, *refs):
    w_refs = refs[:nw]
    b_refs = refs[nw:2 * nw]
    out_ref = refs[2 * nw]
    q = q_ref[0]
    qxyz = q[:, :3]
    qfeat = q[:, 3:3 + fq]
    interp = _knn3_mean(qxyz, st_ref[0], sf_ref[0])
    w0 = w_refs[0][...]
    h = jax.nn.relu((jnp.dot(qfeat, w0[:fq, :]) + jnp.dot(interp, w0[fq:, :])
                     + b_refs[0][...]) * _BN_S)
    for i in range(1, nw):
        h = jax.nn.relu((jnp.dot(h, w_refs[i][...]) + b_refs[i][...]) * _BN_S)
    out_ref[0, :, 0:3] = qxyz
    out_ref[0, :, 3:] = h


def _fp(q, sxyzT, sfeat, Ws, bs, fq, rb):
    B, Nq, Cq = q.shape
    Ns = sfeat.shape[1]
    Fs = sfeat.shape[2]
    nw = len(Ws)
    fout = Ws[-1].shape[1]
    full = lambda s: pl.BlockSpec(s, lambda b, j: (0,) * len(s))
    in_specs = [pl.BlockSpec((1, rb, Cq), lambda b, j: (b, j, 0)),
                pl.BlockSpec((1, 3, Ns), lambda b, j: (b, 0, 0)),
                pl.BlockSpec((1, Ns, Fs), lambda b, j: (b, 0, 0))]
    in_specs += [full(W.shape) for W in Ws]
    in_specs += [full((1, bb.shape[0])) for bb in bs]
    out = pl.pallas_call(
        functools.partial(_fp_body, nw, fq),
        grid=(B, Nq // rb),
        in_specs=in_specs,
        out_specs=pl.BlockSpec((1, rb, 3 + fout), lambda b, j: (b, j, 0)),
        out_shape=jax.ShapeDtypeStruct((B, Nq, 3 + fout), jnp.float32),
    )(q, sxyzT, sfeat, *Ws, *[bb.reshape(1, -1) for bb in bs])
    return out


def _fp_head_body(nw, fq, q_ref, st_ref, sf_ref, *refs):
    w_refs = refs[:nw]
    b_refs = refs[nw:2 * nw]
    hw_ref = refs[2 * nw]
    hb_ref = refs[2 * nw + 1]
    out_ref = refs[2 * nw + 2]
    q = q_ref[0]
    qxyz = q[:, :3]
    qfeat = q[:, 3:3 + fq]
    interp = _knn3_mean(qxyz, st_ref[0], sf_ref[0])
    w0 = w_refs[0][...]
    h = jax.nn.relu((jnp.dot(qfeat, w0[:fq, :]) + jnp.dot(interp, w0[fq:, :])
                     + b_refs[0][...]) * _BN_S)
    for i in range(1, nw):
        h = jax.nn.relu((jnp.dot(h, w_refs[i][...]) + b_refs[i][...]) * _BN_S)
    logits = jnp.dot(h, hw_ref[...]) + hb_ref[...]
    m = jnp.max(logits, axis=1, keepdims=True)
    sh = logits - m
    out_ref[0] = sh - jnp.log(jnp.sum(jnp.exp(sh), axis=1, keepdims=True))


def _fp_head(q, sxyzT, sfeat, Ws, bs, headW, headb, fq, rb):
    B, Nq, Cq = q.shape
    Ns = sfeat.shape[1]
    Fs = sfeat.shape[2]
    nw = len(Ws)
    ncls = headW.shape[1]
    full = lambda s: pl.BlockSpec(s, lambda b, j: (0,) * len(s))
    in_specs = [pl.BlockSpec((1, rb, Cq), lambda b, j: (b, j, 0)),
                pl.BlockSpec((1, 3, Ns), lambda b, j: (b, 0, 0)),
                pl.BlockSpec((1, Ns, Fs), lambda b, j: (b, 0, 0))]
    in_specs += [full(W.shape) for W in Ws]
    in_specs += [full((1, bb.shape[0])) for bb in bs]
    in_specs += [full(headW.shape), full((1, ncls))]
    out = pl.pallas_call(
        functools.partial(_fp_head_body, nw, fq),
        grid=(B, Nq // rb),
        in_specs=in_specs,
        out_specs=pl.BlockSpec((1, rb, ncls), lambda b, j: (b, j, 0)),
        out_shape=jax.ShapeDtypeStruct((B, Nq, ncls), jnp.float32),
    )(q, sxyzT, sfeat, *Ws, *[bb.reshape(1, -1) for bb in bs], headW,
      headb.reshape(1, -1))
    return out


def _gac_layer(x, Ws, bs, a, npoint, radius, nsample, cb_ball, cb_att,
               xyzT=None):
    B, N, _ = x.shape
    f = Ws[-1].shape[1]
    combine = 2 * f <= 256
    tabs = _feat_table(x, Ws, bs, a, combine)
    if xyzT is None:
        xyzT = jnp.transpose(x[..., :3], (0, 2, 1))
    ci, cxyz = _fps(xyzT, npoint)
    gi = _ballq(cxyz, xyzT, radius, nsample, cb_ball)
    gif = gi.reshape(-1)
    cif = ci.reshape(-1)
    if combine:
        tabf = tabs.reshape(B * N, 2 * f)
        grp = _gather_rows(tabf, gif).reshape(B, npoint, nsample, 2 * f)
        cent = _gather_rows(tabf, cif).reshape(B, npoint, 2 * f)
        return _att_comb(grp, cent, cxyz, f, cb_att), cxyz
    feat, ga = tabs
    featf = feat.reshape(B * N, f)
    gaf = ga.reshape(B * N, f)
    gfeat = _gather_rows(featf, gif).reshape(B, npoint, nsample, f)
    gga = _gather_rows(gaf, gif).reshape(B, npoint, nsample, f)
    ca = _gather_rows(gaf, cif).reshape(B, npoint, f)
    return _att(gfeat, gga, ca, cxyz, f, cb_att), cxyz


def kernel(points, params):
    B = points.shape[0]
    l1, cxyz1 = _gac_layer(points, params['l1_W'], params['l1_b'],
                           params['a1'], _NPOINT1, _RADIUS1, _NSAMPLE1,
                           cb_ball=256, cb_att=128)
    l2, _ = _gac_layer(l1, params['l2_W'], params['l2_b'], params['a2'],
                       _NPOINT2, _RADIUS2, _NSAMPLE2, cb_ball=256, cb_att=128,
                       xyzT=jnp.transpose(cxyz1, (0, 2, 1)))
    l2xyzT = jnp.transpose(l2[..., :3], (0, 2, 1))
    l1u = _fp(l1, l2xyzT, l2[..., 3:], params['fp2_W'], params['fp2_b'],
              fq=128, rb=1024)
    l1uxyzT = jnp.transpose(l1u[..., :3], (0, 2, 1))
    logits = _fp_head(points, l1uxyzT, l1u[..., 3:], params['fp1_W'],
                      params['fp1_b'], params['head_W'], params['head_b'],
                      fq=6, rb=1024)
    return logits
